# baseline (speedup 1.0000x reference)
import functools

import jax
import jax.numpy as jnp
from jax import lax
from jax.experimental import pallas as pl
from jax.experimental.pallas import tpu as pltpu
from jax.experimental.pallas import tpu_sc as plsc

_MBLK = 376


def _tc_knn_body(o_ref, h_ref, nnd_ref, nni_ref):
    o = o_ref[0]
    h = h_ref[0]
    g = jnp.dot(o, h, preferred_element_type=jnp.float32)
    o2 = jnp.sum(o * o, axis=1, keepdims=True)
    h2 = jnp.sum(h * h, axis=0, keepdims=True)
    d2 = (o2 + h2) - 2.0 * g
    mn = jnp.min(d2, axis=1, keepdims=True)
    ji = lax.broadcasted_iota(jnp.int32, d2.shape, 1)
    idx = jnp.min(jnp.where(d2 == mn, ji, jnp.int32(2**30)), axis=1)
    nnd_ref[0, 0, :] = mn[:, 0]
    nni_ref[0, 0, :] = idx


def _sc_interior_body(No, hand_h, norm_h, obj_h, nnd_h, nni_h, out_h,
                      hand_v, norm_v, obj_v, nnd_v, nni_v, acc_v):
    w = lax.axis_index("s") * 2 + lax.axis_index("c")
    NhF = hand_v.shape[0]
    NoF = obj_v.shape[0]
    NoP = nnd_v.shape[0]
    pltpu.sync_copy(hand_h.at[pl.ds(w * NhF, NhF)], hand_v)
    pltpu.sync_copy(norm_h.at[pl.ds(w * NhF, NhF)], norm_v)
    pltpu.sync_copy(obj_h.at[pl.ds(w * NoF, NoF)], obj_v)
    pltpu.sync_copy(nnd_h.at[pl.ds(w * NoP, NoP)], nnd_v)
    pltpu.sync_copy(nni_h.at[pl.ds(w * NoP, NoP)], nni_v)
    lane = lax.iota(jnp.int32, 16)

    def body(i, acc):
        st = i * 16
        i3 = nni_v[pl.ds(st, 16)] * 3
        o3 = (st + lane) * 3
        gx = plsc.load_gather(hand_v, [i3])
        gy = plsc.load_gather(hand_v, [i3 + 1])
        gz = plsc.load_gather(hand_v, [i3 + 2])
        nx = plsc.load_gather(norm_v, [i3])
        ny = plsc.load_gather(norm_v, [i3 + 1])
        nz = plsc.load_gather(norm_v, [i3 + 2])
        ox = plsc.load_gather(obj_v, [o3])
        oy = plsc.load_gather(obj_v, [o3 + 1])
        oz = plsc.load_gather(obj_v, [o3 + 2])
        score = (gx - ox) * nx + (gy - oy) * ny + (gz - oz) * nz
        nnd = nnd_v[pl.ds(st, 16)]
        keep = jnp.logical_and(score > 0.0, (st + lane) < No)
        return acc + jnp.where(keep, nnd, 0.0)

    acc = lax.fori_loop(0, NoP // 16, body, jnp.zeros((16,), jnp.float32))
    acc_v[...] = acc
    pltpu.sync_copy(acc_v, out_h.at[pl.ds(w * 16, 16)])


def kernel(hand_xyz, hand_normal, obj_xyz):
    B, Nh, _ = hand_xyz.shape
    No = obj_xyz.shape[1]
    NhP = 896
    NoP = 3008
    NhF = 2336
    NoF = 9024
    nblk = NoP // _MBLK

    f32 = jnp.float32
    hand_t = jnp.transpose(hand_xyz, (0, 2, 1))
    hand_p = jnp.zeros((B, 8, NhP), f32)
    hand_p = hand_p.at[:, :3, :Nh].set(hand_t)
    hand_p = hand_p.at[:, 0, Nh:].set(1e9)
    obj_p = jnp.zeros((B, NoP, 8), f32)
    obj_p = obj_p.at[:, :No, :3].set(obj_xyz)

    nnd, nni = pl.pallas_call(
        _tc_knn_body,
        grid=(B, nblk),
        in_specs=[
            pl.BlockSpec((1, _MBLK, 8), lambda b, m: (b, m, 0)),
            pl.BlockSpec((1, 8, NhP), lambda b, m: (b, 0, 0)),
        ],
        out_specs=[
            pl.BlockSpec((1, 1, _MBLK), lambda b, m: (b * nblk + m, 0, 0)),
            pl.BlockSpec((1, 1, _MBLK), lambda b, m: (b * nblk + m, 0, 0)),
        ],
        out_shape=[
            jax.ShapeDtypeStruct((B * nblk, 1, _MBLK), f32),
            jax.ShapeDtypeStruct((B * nblk, 1, _MBLK), jnp.int32),
        ],
        compiler_params=pltpu.CompilerParams(
            dimension_semantics=("parallel", "parallel")),
    )(obj_p, hand_p)

    hand_f = jnp.pad(hand_xyz.reshape(B, Nh * 3), ((0, 0), (0, NhF - Nh * 3)))
    norm_f = jnp.pad(hand_normal.reshape(B, Nh * 3),
                     ((0, 0), (0, NhF - Nh * 3)))
    obj_f = jnp.pad(obj_xyz.reshape(B, No * 3), ((0, 0), (0, NoF - No * 3)))

    mesh = plsc.VectorSubcoreMesh(core_axis_name="c", subcore_axis_name="s")
    sc = functools.partial(
        pl.kernel,
        mesh=mesh,
        compiler_params=pltpu.CompilerParams(needs_layout_passes=False),
        out_type=jax.ShapeDtypeStruct((B * 16,), f32),
        scratch_types=[
            pltpu.VMEM((NhF,), f32),
            pltpu.VMEM((NhF,), f32),
            pltpu.VMEM((NoF,), f32),
            pltpu.VMEM((NoP,), f32),
            pltpu.VMEM((NoP,), jnp.int32),
            pltpu.VMEM((16,), f32),
        ],
    )(functools.partial(_sc_interior_body, No))
    partials = sc(hand_f.reshape(-1), norm_f.reshape(-1), obj_f.reshape(-1),
                  nnd.reshape(-1), nni.reshape(-1))

    return jnp.sum(partials) / B


# speedup vs baseline: 1.0917x; 1.0917x over previous
import functools

import jax
import jax.numpy as jnp
from jax import lax
from jax.experimental import pallas as pl
from jax.experimental.pallas import tpu as pltpu
from jax.experimental.pallas import tpu_sc as plsc

_MBLK = 376


def _tc_knn_body(o_ref, h_ref, nnd_ref, nni_ref):
    o = o_ref[0]
    h = h_ref[0]
    g = jnp.dot(o, h, preferred_element_type=jnp.float32)
    o2 = jnp.sum(o * o, axis=1, keepdims=True)
    h2 = jnp.sum(h * h, axis=0, keepdims=True)
    d2 = (o2 + h2) - 2.0 * g
    mn = jnp.min(d2, axis=1, keepdims=True)
    ji = lax.broadcasted_iota(jnp.int32, (1, d2.shape[1]), 1).astype(
        jnp.float32)
    idxf = jnp.min(jnp.where(d2 == mn, ji, jnp.float32(2**30)), axis=1,
                   keepdims=True)
    nnd_ref[0] = mn
    nni_ref[0] = idxf.astype(jnp.int32)


def _sc_interior_body(No, hand_h, norm_h, obj_h, nnd_h, nni_h, out_h,
                      hand_v, norm_v, obj_v, nnd_v, nni_v, acc_v):
    w = lax.axis_index("s") * 2 + lax.axis_index("c")
    NhF = hand_v.shape[0]
    NoF = obj_v.shape[0]
    NoP = nnd_v.shape[0]
    pltpu.sync_copy(hand_h.at[pl.ds(w * NhF, NhF)], hand_v)
    pltpu.sync_copy(norm_h.at[pl.ds(w * NhF, NhF)], norm_v)
    pltpu.sync_copy(obj_h.at[pl.ds(w * NoF, NoF)], obj_v)
    pltpu.sync_copy(nnd_h.at[pl.ds(w * NoP, NoP)], nnd_v)
    pltpu.sync_copy(nni_h.at[pl.ds(w * NoP, NoP)], nni_v)
    lane = lax.iota(jnp.int32, 16)

    def body(i, acc):
        st = i * 16
        i3 = nni_v[pl.ds(st, 16)] * 3
        o3 = (st + lane) * 3
        gx = plsc.load_gather(hand_v, [i3])
        gy = plsc.load_gather(hand_v, [i3 + 1])
        gz = plsc.load_gather(hand_v, [i3 + 2])
        nx = plsc.load_gather(norm_v, [i3])
        ny = plsc.load_gather(norm_v, [i3 + 1])
        nz = plsc.load_gather(norm_v, [i3 + 2])
        ox = plsc.load_gather(obj_v, [o3])
        oy = plsc.load_gather(obj_v, [o3 + 1])
        oz = plsc.load_gather(obj_v, [o3 + 2])
        score = (gx - ox) * nx + (gy - oy) * ny + (gz - oz) * nz
        nnd = nnd_v[pl.ds(st, 16)]
        keep = jnp.logical_and(score > 0.0, (st + lane) < No)
        return acc + jnp.where(keep, nnd, 0.0)

    acc = lax.fori_loop(0, NoP // 16, body, jnp.zeros((16,), jnp.float32))
    acc_v[...] = acc
    pltpu.sync_copy(acc_v, out_h.at[pl.ds(w * 16, 16)])


def kernel(hand_xyz, hand_normal, obj_xyz):
    B, Nh, _ = hand_xyz.shape
    No = obj_xyz.shape[1]
    NhP = 896
    NoP = 3008
    NhF = 2336
    NoF = 9024
    nblk = NoP // _MBLK

    f32 = jnp.float32
    hand_t = jnp.transpose(hand_xyz, (0, 2, 1))
    hand_p = jnp.zeros((B, 8, NhP), f32)
    hand_p = hand_p.at[:, :3, :Nh].set(hand_t)
    hand_p = hand_p.at[:, 0, Nh:].set(1e9)
    obj_p = jnp.zeros((B, NoP, 8), f32)
    obj_p = obj_p.at[:, :No, :3].set(obj_xyz)

    nnd, nni = pl.pallas_call(
        _tc_knn_body,
        grid=(B, nblk),
        in_specs=[
            pl.BlockSpec((1, _MBLK, 8), lambda b, m: (b, m, 0)),
            pl.BlockSpec((1, 8, NhP), lambda b, m: (b, 0, 0)),
        ],
        out_specs=[
            pl.BlockSpec((1, _MBLK, 1), lambda b, m: (b * nblk + m, 0, 0)),
            pl.BlockSpec((1, _MBLK, 1), lambda b, m: (b * nblk + m, 0, 0)),
        ],
        out_shape=[
            jax.ShapeDtypeStruct((B * nblk, _MBLK, 1), f32),
            jax.ShapeDtypeStruct((B * nblk, _MBLK, 1), jnp.int32),
        ],
        compiler_params=pltpu.CompilerParams(
            dimension_semantics=("parallel", "parallel")),
    )(obj_p, hand_p)

    hand_f = jnp.pad(hand_xyz.reshape(B, Nh * 3), ((0, 0), (0, NhF - Nh * 3)))
    norm_f = jnp.pad(hand_normal.reshape(B, Nh * 3),
                     ((0, 0), (0, NhF - Nh * 3)))
    obj_f = jnp.pad(obj_xyz.reshape(B, No * 3), ((0, 0), (0, NoF - No * 3)))

    mesh = plsc.VectorSubcoreMesh(core_axis_name="c", subcore_axis_name="s")
    sc = functools.partial(
        pl.kernel,
        mesh=mesh,
        compiler_params=pltpu.CompilerParams(needs_layout_passes=False),
        out_type=jax.ShapeDtypeStruct((B * 16,), f32),
        scratch_types=[
            pltpu.VMEM((NhF,), f32),
            pltpu.VMEM((NhF,), f32),
            pltpu.VMEM((NoF,), f32),
            pltpu.VMEM((NoP,), f32),
            pltpu.VMEM((NoP,), jnp.int32),
            pltpu.VMEM((16,), f32),
        ],
    )(functools.partial(_sc_interior_body, No))
    partials = sc(hand_f.reshape(-1), norm_f.reshape(-1), obj_f.reshape(-1),
                  nnd.reshape(-1), nni.reshape(-1))

    return jnp.sum(partials) / B


# speedup vs baseline: 1.4297x; 1.3097x over previous
import functools

import jax
import jax.numpy as jnp
from jax import lax
from jax.experimental import pallas as pl
from jax.experimental.pallas import tpu as pltpu
from jax.experimental.pallas import tpu_sc as plsc

_MBLK = 384


def _tc_knn_body(o_ref, h_ref, nnd_ref, nni_ref):
    o = o_ref[0]
    h = h_ref[0]
    cdims = (((0,), (0,)), ((), ()))
    g = lax.dot_general(h, o, cdims,
                        preferred_element_type=jnp.float32)
    o2 = jnp.sum(o * o, axis=0, keepdims=True)
    h2 = lax.dot_general(h * h, jnp.ones((8, 1), jnp.float32), cdims,
                         preferred_element_type=jnp.float32)
    d2 = (o2 + h2) - 2.0 * g
    mn = jnp.min(d2, axis=0, keepdims=True)
    ji = lax.broadcasted_iota(jnp.int32, (d2.shape[0], 1), 0).astype(
        jnp.float32)
    idxf = jnp.min(jnp.where(d2 == mn, ji, jnp.float32(2**30)), axis=0,
                   keepdims=True)
    nnd_ref[0] = mn
    nni_ref[0] = idxf.astype(jnp.int32)


def _sc_interior_body(No, hand_h, norm_h, obj_h, nnd_h, nni_h, out_h,
                      hand_v, norm_v, obj_v, nnd_v, nni_v, acc_v):
    w = lax.axis_index("s") * 2 + lax.axis_index("c")
    NhF = hand_v.shape[0]
    NoF = obj_v.shape[0]
    NoP = nnd_v.shape[0]
    pltpu.sync_copy(hand_h.at[pl.ds(w * NhF, NhF)], hand_v)
    pltpu.sync_copy(norm_h.at[pl.ds(w * NhF, NhF)], norm_v)
    pltpu.sync_copy(obj_h.at[pl.ds(w * NoF, NoF)], obj_v)
    pltpu.sync_copy(nnd_h.at[pl.ds(w * NoP, NoP)], nnd_v)
    pltpu.sync_copy(nni_h.at[pl.ds(w * NoP, NoP)], nni_v)
    lane = lax.iota(jnp.int32, 16)

    def body(i, acc):
        st = i * 16
        i3 = nni_v[pl.ds(st, 16)] * 3
        o3 = (st + lane) * 3
        gx = plsc.load_gather(hand_v, [i3])
        gy = plsc.load_gather(hand_v, [i3 + 1])
        gz = plsc.load_gather(hand_v, [i3 + 2])
        nx = plsc.load_gather(norm_v, [i3])
        ny = plsc.load_gather(norm_v, [i3 + 1])
        nz = plsc.load_gather(norm_v, [i3 + 2])
        ox = plsc.load_gather(obj_v, [o3])
        oy = plsc.load_gather(obj_v, [o3 + 1])
        oz = plsc.load_gather(obj_v, [o3 + 2])
        score = (gx - ox) * nx + (gy - oy) * ny + (gz - oz) * nz
        nnd = nnd_v[pl.ds(st, 16)]
        keep = jnp.logical_and(score > 0.0, (st + lane) < No)
        return acc + jnp.where(keep, nnd, 0.0)

    acc = lax.fori_loop(0, NoP // 16, body, jnp.zeros((16,), jnp.float32))
    acc_v[...] = acc
    pltpu.sync_copy(acc_v, out_h.at[pl.ds(w * 16, 16)])


def kernel(hand_xyz, hand_normal, obj_xyz):
    B, Nh, _ = hand_xyz.shape
    No = obj_xyz.shape[1]
    NhP = 896
    NoP = 3072
    NhF = 2336
    NoF = 9024
    nblk = NoP // _MBLK

    f32 = jnp.float32
    hand_t = jnp.transpose(hand_xyz, (0, 2, 1))
    hand_p = jnp.zeros((B, 8, NhP), f32)
    hand_p = hand_p.at[:, :3, :Nh].set(hand_t)
    hand_p = hand_p.at[:, 0, Nh:].set(1e9)
    obj_t = jnp.transpose(obj_xyz, (0, 2, 1))
    obj_p = jnp.zeros((B, 8, NoP), f32)
    obj_p = obj_p.at[:, :3, :No].set(obj_t)

    nnd, nni = pl.pallas_call(
        _tc_knn_body,
        grid=(B, nblk),
        in_specs=[
            pl.BlockSpec((1, 8, _MBLK), lambda b, m: (b, 0, m)),
            pl.BlockSpec((1, 8, NhP), lambda b, m: (b, 0, 0)),
        ],
        out_specs=[
            pl.BlockSpec((1, 1, _MBLK), lambda b, m: (b * nblk + m, 0, 0)),
            pl.BlockSpec((1, 1, _MBLK), lambda b, m: (b * nblk + m, 0, 0)),
        ],
        out_shape=[
            jax.ShapeDtypeStruct((B * nblk, 1, _MBLK), f32),
            jax.ShapeDtypeStruct((B * nblk, 1, _MBLK), jnp.int32),
        ],
        compiler_params=pltpu.CompilerParams(
            dimension_semantics=("parallel", "parallel")),
    )(obj_p, hand_p)

    hand_f = jnp.pad(hand_xyz.reshape(B, Nh * 3), ((0, 0), (0, NhF - Nh * 3)))
    norm_f = jnp.pad(hand_normal.reshape(B, Nh * 3),
                     ((0, 0), (0, NhF - Nh * 3)))
    obj_f = jnp.pad(obj_xyz.reshape(B, No * 3), ((0, 0), (0, NoF - No * 3)))

    mesh = plsc.VectorSubcoreMesh(core_axis_name="c", subcore_axis_name="s")
    sc = functools.partial(
        pl.kernel,
        mesh=mesh,
        compiler_params=pltpu.CompilerParams(needs_layout_passes=False),
        out_type=jax.ShapeDtypeStruct((B * 16,), f32),
        scratch_types=[
            pltpu.VMEM((NhF,), f32),
            pltpu.VMEM((NhF,), f32),
            pltpu.VMEM((NoF,), f32),
            pltpu.VMEM((NoP,), f32),
            pltpu.VMEM((NoP,), jnp.int32),
            pltpu.VMEM((16,), f32),
        ],
    )(functools.partial(_sc_interior_body, No))
    partials = sc(hand_f.reshape(-1), norm_f.reshape(-1), obj_f.reshape(-1),
                  nnd.reshape(-1), nni.reshape(-1))

    return jnp.sum(partials) / B
